# baseline (speedup 1.0000x reference)
import jax
import jax.numpy as jnp
from jax.experimental import pallas as pl
from jax.experimental.pallas import tpu as pltpu

BM = 200
BK = 3584
HEAD = 128


def _mlp_kernel(x_ref, w1_ref, b1_ref, w2_ref, b2_ref, wh_ref, bh_ref,
                out_ref, acc_ref):
    k = pl.program_id(1)
    nk = pl.num_programs(1)

    @pl.when(k == 0)
    def _():
        acc_ref[...] = jnp.zeros_like(acc_ref)

    acc_ref[...] += jnp.dot(x_ref[...], w1_ref[...],
                            preferred_element_type=jnp.float32)

    @pl.when(k == nk - 1)
    def _():
        h1 = jnp.maximum(acc_ref[...] + b1_ref[...], 0.0)
        h2 = jnp.maximum(
            jnp.dot(h1, w2_ref[...], preferred_element_type=jnp.float32)
            + b2_ref[...], 0.0)
        out_ref[...] = (jnp.dot(h2, wh_ref[...],
                                preferred_element_type=jnp.float32)
                        + bh_ref[...])


def kernel(feature_vectors, W1, b1, W2, b2, Wc, bc, Wr, br):
    n, d_in = feature_vectors.shape
    hid = W1.shape[1]
    nc = Wc.shape[1]
    nr = Wr.shape[1]

    wh = jnp.pad(jnp.concatenate([Wc, Wr], axis=1),
                 ((0, 0), (0, HEAD - nc - nr)))
    bh = jnp.pad(jnp.concatenate([bc, br]), (0, HEAD - nc - nr)).reshape(1, HEAD)
    b1r = b1.reshape(1, hid)
    b2r = b2.reshape(1, hid)

    grid = (n // BM, d_in // BK)
    out = pl.pallas_call(
        _mlp_kernel,
        grid=grid,
        in_specs=[
            pl.BlockSpec((BM, BK), lambda i, k: (i, k)),
            pl.BlockSpec((BK, hid), lambda i, k: (k, 0)),
            pl.BlockSpec((1, hid), lambda i, k: (0, 0)),
            pl.BlockSpec((hid, hid), lambda i, k: (0, 0)),
            pl.BlockSpec((1, hid), lambda i, k: (0, 0)),
            pl.BlockSpec((hid, HEAD), lambda i, k: (0, 0)),
            pl.BlockSpec((1, HEAD), lambda i, k: (0, 0)),
        ],
        out_specs=pl.BlockSpec((BM, HEAD), lambda i, k: (i, 0)),
        out_shape=jax.ShapeDtypeStruct((n, HEAD), jnp.float32),
        scratch_shapes=[pltpu.VMEM((BM, hid), jnp.float32)],
        compiler_params=pltpu.CompilerParams(
            dimension_semantics=("parallel", "arbitrary"),
        ),
    )(feature_vectors, W1, b1r, W2, b2r, wh, bh)
    return out[:, :nc], out[:, nc:nc + nr]


# speedup vs baseline: 1.8105x; 1.8105x over previous
import jax
import jax.numpy as jnp
from jax.experimental import pallas as pl
from jax.experimental.pallas import tpu as pltpu

BM = 200
BK = 3584
HEAD = 128


def _mlp_kernel(x_ref, w1_ref, b1_ref, w2_ref, b2_ref, wh_ref, bh_ref,
                out_ref, acc_ref):
    k = pl.program_id(0)
    i = pl.program_id(1)
    nk = pl.num_programs(0)

    @pl.when(k == 0)
    def _():
        acc_ref[i] = jnp.zeros_like(acc_ref[i])

    acc_ref[i] += jnp.dot(x_ref[...], w1_ref[...],
                          preferred_element_type=jnp.float32)

    @pl.when(k == nk - 1)
    def _():
        h1 = jnp.maximum(acc_ref[i] + b1_ref[...], 0.0)
        h2 = jnp.maximum(
            jnp.dot(h1, w2_ref[...], preferred_element_type=jnp.float32)
            + b2_ref[...], 0.0)
        out_ref[...] = (jnp.dot(h2, wh_ref[...],
                                preferred_element_type=jnp.float32)
                        + bh_ref[...])


def kernel(feature_vectors, W1, b1, W2, b2, Wc, bc, Wr, br):
    n, d_in = feature_vectors.shape
    hid = W1.shape[1]
    nc = Wc.shape[1]
    nr = Wr.shape[1]

    wh = jnp.pad(jnp.concatenate([Wc, Wr], axis=1),
                 ((0, 0), (0, HEAD - nc - nr)))
    bh = jnp.pad(jnp.concatenate([bc, br]), (0, HEAD - nc - nr)).reshape(1, HEAD)
    b1r = b1.reshape(1, hid)
    b2r = b2.reshape(1, hid)

    nm = n // BM
    grid = (d_in // BK, nm)
    out = pl.pallas_call(
        _mlp_kernel,
        grid=grid,
        in_specs=[
            pl.BlockSpec((BM, BK), lambda k, i: (i, k)),
            pl.BlockSpec((BK, hid), lambda k, i: (k, 0)),
            pl.BlockSpec((1, hid), lambda k, i: (0, 0)),
            pl.BlockSpec((hid, hid), lambda k, i: (0, 0)),
            pl.BlockSpec((1, hid), lambda k, i: (0, 0)),
            pl.BlockSpec((hid, HEAD), lambda k, i: (0, 0)),
            pl.BlockSpec((1, HEAD), lambda k, i: (0, 0)),
        ],
        out_specs=pl.BlockSpec((BM, HEAD), lambda k, i: (i, 0)),
        out_shape=jax.ShapeDtypeStruct((n, HEAD), jnp.float32),
        scratch_shapes=[pltpu.VMEM((nm, BM, hid), jnp.float32)],
        compiler_params=pltpu.CompilerParams(
            dimension_semantics=("arbitrary", "arbitrary"),
        ),
    )(feature_vectors, W1, b1r, W2, b2r, wh, bh)
    return out[:, :nc], out[:, nc:nc + nr]
